# baseline (speedup 1.0000x reference)
import functools

import jax
import jax.numpy as jnp
from jax.experimental import pallas as pl
from jax.experimental.pallas import tpu as pltpu

TOP_K = 2
EXPLORATION_BONUS = 0.1


def _body(x_ref, w1t_ref, b1_ref, w2_ref, b2_ref, s1t_ref, s1b_ref,
          s2t_ref, s2b_ref, we_ref, be_ref, pred_ref, rw_ref,
          w1_scr, s1_scr, s2_scr, web_scr, hacc, pacc,
          *, n_experts, n_dchunks):
    j = pl.program_id(1)
    dc = x_ref.shape[1]
    joff = j * dc

    @pl.when(pl.program_id(0) == 0)
    def _prep():
        w1_scr[pl.ds(joff, dc), :] = w1t_ref[...].T
        web_scr[pl.ds(joff, dc), :] = we_ref[...].T

        @pl.when(j == 0)
        def _prep_once():
            s1_scr[...] = s1t_ref[...].T
            s2_scr[...] = s2t_ref[...].T

    xt = x_ref[...]
    h_part = jnp.dot(xt, w1_scr[pl.ds(joff, dc), :],
                     preferred_element_type=jnp.float32)
    p_part = jnp.dot(xt, web_scr[pl.ds(joff, dc), :],
                     preferred_element_type=jnp.float32)

    @pl.when(j == 0)
    def _init():
        hacc[...] = h_part
        pacc[...] = p_part

    @pl.when(j > 0)
    def _acc():
        hacc[...] += h_part
        pacc[...] += p_part

    @pl.when(j == n_dchunks - 1)
    def _finish():
        h = jnp.maximum(hacc[...] + b1_ref[...].reshape(1, -1), 0.0)
        ctx = jnp.tanh(
            jnp.dot(h, w2_ref[...], preferred_element_type=jnp.float32)
            + b2_ref[...].reshape(1, -1))
        sh = jnp.maximum(
            jnp.dot(ctx, s1_scr[...], preferred_element_type=jnp.float32)
            + s1b_ref[...].reshape(1, -1), 0.0)
        scores = (jnp.dot(sh, s2_scr[...], preferred_element_type=jnp.float32)
                  + s2b_ref[...].reshape(1, -1) + EXPLORATION_BONUS)
        preds = pacc[...] + be_ref[...].reshape(1, -1)

        scores_t = scores.T
        preds_t = preds.T

        eidx = jax.lax.broadcasted_iota(jnp.int32, scores_t.shape, 0).astype(
            jnp.float32)
        m1 = jnp.max(scores_t, axis=0, keepdims=True)
        i1 = jnp.min(jnp.where(scores_t == m1, eidx, float(n_experts)),
                     axis=0, keepdims=True)
        masked = jnp.where(eidx == i1, -jnp.inf, scores_t)
        m2 = jnp.max(masked, axis=0, keepdims=True)
        i2 = jnp.min(jnp.where(masked == m2, eidx, float(n_experts)),
                     axis=0, keepdims=True)

        e2 = jnp.exp(m2 - m1)
        denom = 1.0 + e2
        w1v = 1.0 / denom
        w2v = e2 / denom

        sel = (jnp.where(eidx == i1, w1v, 0.0)
               + jnp.where(eidx == i2, w2v, 0.0))
        pred_ref[...] = jnp.sum(sel * preds_t, axis=0, keepdims=True)
        rw_ref[...] = jnp.concatenate([w1v, w2v], axis=0)


def kernel(x, W1, b1, W2, b2, S1, s1, S2, s2, We, be):
    n, d = x.shape
    e = S2.shape[1]
    hid1 = W1.shape[1]
    ctxd = W2.shape[1]
    hid2 = S1.shape[1]

    tile = 16384
    dchunks = 6
    dc = d // dchunks
    grid = (n // tile, dchunks)
    c1 = lambda i, j: (0,)
    c2 = lambda i, j: (0, 0)

    preds, rw = pl.pallas_call(
        functools.partial(_body, n_experts=e, n_dchunks=dchunks),
        grid=grid,
        in_specs=[
            pl.BlockSpec((tile, dc), lambda i, j: (i, j)),
            pl.BlockSpec((hid1, dc), lambda i, j: (0, j)),
            pl.BlockSpec((hid1,), c1),
            pl.BlockSpec((hid1, ctxd), c2),
            pl.BlockSpec((ctxd,), c1),
            pl.BlockSpec((hid2, ctxd), c2),
            pl.BlockSpec((hid2,), c1),
            pl.BlockSpec((e, hid2), c2),
            pl.BlockSpec((e,), c1),
            pl.BlockSpec((e, dc), lambda i, j: (0, j)),
            pl.BlockSpec((e,), c1),
        ],
        out_specs=[
            pl.BlockSpec((1, tile), lambda i, j: (0, i)),
            pl.BlockSpec((TOP_K, tile), lambda i, j: (0, i)),
        ],
        out_shape=[
            jax.ShapeDtypeStruct((1, n), jnp.float32),
            jax.ShapeDtypeStruct((TOP_K, n), jnp.float32),
        ],
        scratch_shapes=[
            pltpu.VMEM((d, hid1), jnp.float32),
            pltpu.VMEM((ctxd, hid2), jnp.float32),
            pltpu.VMEM((hid2, e), jnp.float32),
            pltpu.VMEM((d, e), jnp.float32),
            pltpu.VMEM((tile, hid1), jnp.float32),
            pltpu.VMEM((tile, e), jnp.float32),
        ],
    )(x, W1.T, b1, W2, b2, S1.T, s1, S2.T, s2, We[:, :, 0], be.reshape(e))
    return (preds.reshape(n, 1), rw.T)


# speedup vs baseline: 1.2297x; 1.2297x over previous
import functools

import jax
import jax.numpy as jnp
from jax.experimental import pallas as pl
from jax.experimental.pallas import tpu as pltpu

TOP_K = 2
EXPLORATION_BONUS = 0.1


def _body(x_ref, w1t_ref, b1_ref, w2_ref, b2_ref, s1t_ref, s1b_ref,
          s2t_ref, s2b_ref, we_ref, be_ref, pred_ref, rw_ref,
          w1_scr, s1_scr, s2_scr, web_scr, hacc, pacc,
          *, n_experts, n_dchunks):
    j = pl.program_id(1)
    dc = x_ref.shape[1]
    joff = j * dc

    @pl.when(pl.program_id(0) == 0)
    def _prep():
        w1_scr[pl.ds(joff, dc), :] = w1t_ref[...].T
        web_scr[pl.ds(joff, dc), :] = we_ref[...].T

        @pl.when(j == 0)
        def _prep_once():
            s1_scr[...] = s1t_ref[...].T
            s2_scr[...] = s2t_ref[...].T

    xt = x_ref[...]
    h_part = jnp.dot(xt, w1_scr[pl.ds(joff, dc), :],
                     preferred_element_type=jnp.float32)
    p_part = jnp.dot(xt, web_scr[pl.ds(joff, dc), :],
                     preferred_element_type=jnp.float32)

    @pl.when(j == 0)
    def _init():
        hacc[...] = h_part
        pacc[...] = p_part

    @pl.when(j > 0)
    def _acc():
        hacc[...] += h_part
        pacc[...] += p_part

    @pl.when(j == n_dchunks - 1)
    def _finish():
        h = jnp.maximum(hacc[...] + b1_ref[...].reshape(1, -1), 0.0)
        ctx = jnp.tanh(
            jnp.dot(h, w2_ref[...], preferred_element_type=jnp.float32)
            + b2_ref[...].reshape(1, -1))
        sh = jnp.maximum(
            jnp.dot(ctx, s1_scr[...], preferred_element_type=jnp.float32)
            + s1b_ref[...].reshape(1, -1), 0.0)
        scores = (jnp.dot(sh, s2_scr[...], preferred_element_type=jnp.float32)
                  + s2b_ref[...].reshape(1, -1) + EXPLORATION_BONUS)
        preds = pacc[...] + be_ref[...].reshape(1, -1)

        scores_t = scores.T
        preds_t = preds.T

        eidx = jax.lax.broadcasted_iota(jnp.int32, scores_t.shape, 0).astype(
            jnp.float32)
        m1 = jnp.max(scores_t, axis=0, keepdims=True)
        i1 = jnp.min(jnp.where(scores_t == m1, eidx, float(n_experts)),
                     axis=0, keepdims=True)
        masked = jnp.where(eidx == i1, -jnp.inf, scores_t)
        m2 = jnp.max(masked, axis=0, keepdims=True)
        i2 = jnp.min(jnp.where(masked == m2, eidx, float(n_experts)),
                     axis=0, keepdims=True)

        e2 = jnp.exp(m2 - m1)
        denom = 1.0 + e2
        w1v = 1.0 / denom
        w2v = e2 / denom

        sel = (jnp.where(eidx == i1, w1v, 0.0)
               + jnp.where(eidx == i2, w2v, 0.0))
        pred_ref[...] = jnp.sum(sel * preds_t, axis=0, keepdims=True)
        rw_ref[...] = jnp.concatenate([w1v, w2v], axis=0)


def kernel(x, W1, b1, W2, b2, S1, s1, S2, s2, We, be):
    n, d = x.shape
    e = S2.shape[1]
    hid1 = W1.shape[1]
    ctxd = W2.shape[1]
    hid2 = S1.shape[1]

    tile = 8192
    dchunks = 2
    dc = d // dchunks
    grid = (n // tile, dchunks)
    c1 = lambda i, j: (0,)
    c2 = lambda i, j: (0, 0)

    preds, rw = pl.pallas_call(
        functools.partial(_body, n_experts=e, n_dchunks=dchunks),
        grid=grid,
        in_specs=[
            pl.BlockSpec((tile, dc), lambda i, j: (i, j)),
            pl.BlockSpec((hid1, dc), lambda i, j: (0, j)),
            pl.BlockSpec((hid1,), c1),
            pl.BlockSpec((hid1, ctxd), c2),
            pl.BlockSpec((ctxd,), c1),
            pl.BlockSpec((hid2, ctxd), c2),
            pl.BlockSpec((hid2,), c1),
            pl.BlockSpec((e, hid2), c2),
            pl.BlockSpec((e,), c1),
            pl.BlockSpec((e, dc), lambda i, j: (0, j)),
            pl.BlockSpec((e,), c1),
        ],
        out_specs=[
            pl.BlockSpec((1, tile), lambda i, j: (0, i)),
            pl.BlockSpec((TOP_K, tile), lambda i, j: (0, i)),
        ],
        out_shape=[
            jax.ShapeDtypeStruct((1, n), jnp.float32),
            jax.ShapeDtypeStruct((TOP_K, n), jnp.float32),
        ],
        scratch_shapes=[
            pltpu.VMEM((d, hid1), jnp.float32),
            pltpu.VMEM((ctxd, hid2), jnp.float32),
            pltpu.VMEM((hid2, e), jnp.float32),
            pltpu.VMEM((d, e), jnp.float32),
            pltpu.VMEM((tile, hid1), jnp.float32),
            pltpu.VMEM((tile, e), jnp.float32),
        ],
    )(x, W1.T, b1, W2, b2, S1.T, s1, S2.T, s2, We[:, :, 0], be.reshape(e))
    return (preds.reshape(n, 1), rw.T)
